# baseline (speedup 1.0000x reference)
import functools
import jax
import jax.numpy as jnp
from jax import lax
from jax.experimental import pallas as pl
from jax.experimental.pallas import tpu as pltpu
from jax.experimental.pallas import tpu_sc as plsc

_NC = 2
_NS = 16
_NW = _NC * _NS
_L = 16
_NBUF = 2


def kernel(x, pos_embed_weight):
    S, B, D = x.shape
    pe = pos_embed_weight[:S]
    rows_per_w = S // _NW
    CS = 8
    n_chunks = rows_per_w // CS
    n_groups = n_chunks // _NBUF
    nvec = D // _L

    mesh = plsc.VectorSubcoreMesh(core_axis_name="c", subcore_axis_name="s")

    scratch = (
        [pltpu.VMEM((CS, B, D), jnp.float32) for _ in range(_NBUF)]
        + [pltpu.VMEM((CS, D), jnp.float32) for _ in range(_NBUF)]
        + [pltpu.SemaphoreType.DMA for _ in range(3 * _NBUF)]
    )

    @functools.partial(
        pl.kernel,
        mesh=mesh,
        out_type=jax.ShapeDtypeStruct((S, B, D), jnp.float32),
        scratch_types=scratch,
    )
    def k(x_hbm, pe_hbm, out_hbm, *bufs):
        xbufs = bufs[0:_NBUF]
        pbufs = bufs[_NBUF:2 * _NBUF]
        six = bufs[2 * _NBUF:3 * _NBUF]
        sip = bufs[3 * _NBUF:4 * _NBUF]
        so = bufs[4 * _NBUF:5 * _NBUF]
        wid = lax.axis_index("s") * _NC + lax.axis_index("c")
        base = wid * rows_per_w

        def in_copy(ci, b):
            r0 = base + ci * CS
            return (
                pltpu.make_async_copy(x_hbm.at[pl.ds(r0, CS)], xbufs[b], six[b]),
                pltpu.make_async_copy(pe_hbm.at[pl.ds(r0, CS)], pbufs[b], sip[b]),
            )

        def out_copy(ci, b):
            r0 = base + ci * CS
            return pltpu.make_async_copy(xbufs[b], out_hbm.at[pl.ds(r0, CS)], so[b])

        def compute(b):
            xb_ = xbufs[b]
            pb_ = pbufs[b]

            def row_body(r, _):
                @plsc.parallel_loop(0, nvec, unroll=8)
                def vec_body(j):
                    sl = pl.ds(j * _L, _L)
                    pev = pb_[r, sl]
                    for bb in range(B):
                        xb_[r, bb, sl] = xb_[r, bb, sl] + pev

                return 0

            lax.fori_loop(0, CS, row_body, 0)

        for b in range(_NBUF):
            hx, hp = in_copy(b, b)
            hx.start()
            hp.start()

        def group_body(g, _):
            ci0 = g * _NBUF
            for b in range(_NBUF):
                ci = ci0 + b
                hx, hp = in_copy(ci, b)
                hx.wait()
                hp.wait()
                compute(b)
                out_copy(ci, b).start()

            @pl.when(g < n_groups - 1)
            def _prefetch():
                for b in range(_NBUF):
                    ci = ci0 + b
                    out_copy(ci, b).wait()
                    hx, hp = in_copy(ci + _NBUF, b)
                    hx.start()
                    hp.start()

            return 0

        lax.fori_loop(0, n_groups, group_body, 0)
        for b in range(_NBUF):
            out_copy(n_chunks - _NBUF + b, b).wait()

    return k(x, pe)


# speedup vs baseline: 1.0196x; 1.0196x over previous
import functools
import jax
import jax.numpy as jnp
from jax import lax
from jax.experimental import pallas as pl
from jax.experimental.pallas import tpu as pltpu
from jax.experimental.pallas import tpu_sc as plsc

_NC = 2
_NS = 16
_NW = _NC * _NS
_L = 16
_NBUF = 3


def kernel(x, pos_embed_weight):
    S, B, D = x.shape
    pe = pos_embed_weight[:S]
    rows_per_w = S // _NW
    CS = 8
    n_chunks = rows_per_w // CS
    nvec = D // _L

    mesh = plsc.VectorSubcoreMesh(core_axis_name="c", subcore_axis_name="s")

    scratch = (
        [pltpu.VMEM((CS, B, D), jnp.float32) for _ in range(_NBUF)]
        + [pltpu.VMEM((CS, D), jnp.float32) for _ in range(_NBUF)]
        + [pltpu.SemaphoreType.DMA for _ in range(3 * _NBUF)]
    )

    @functools.partial(
        pl.kernel,
        mesh=mesh,
        out_type=jax.ShapeDtypeStruct((S, B, D), jnp.float32),
        scratch_types=scratch,
    )
    def k(x_hbm, pe_hbm, out_hbm, *bufs):
        xbufs = bufs[0:_NBUF]
        pbufs = bufs[_NBUF:2 * _NBUF]
        six = bufs[2 * _NBUF:3 * _NBUF]
        sip = bufs[3 * _NBUF:4 * _NBUF]
        so = bufs[4 * _NBUF:5 * _NBUF]
        wid = lax.axis_index("s") * _NC + lax.axis_index("c")
        base = wid * rows_per_w

        def issue_in(ci):
            p = ci % _NBUF
            r0 = base + ci * CS
            hx = pltpu.async_copy(x_hbm.at[pl.ds(r0, CS)], xbufs[p], six[p])
            hp = pltpu.async_copy(pe_hbm.at[pl.ds(r0, CS)], pbufs[p], sip[p])
            return hx, hp

        def compute(p):
            xb_ = xbufs[p]
            pb_ = pbufs[p]

            def row_body(r, _):
                @plsc.parallel_loop(0, nvec, unroll=16)
                def vec_body(j):
                    sl = pl.ds(j * _L, _L)
                    pev = pb_[r, sl]
                    for b in range(B):
                        xb_[r, b, sl] = xb_[r, b, sl] + pev

                return 0

            lax.fori_loop(0, CS, row_body, 0)

        hin = {}
        hout = {}
        for ci in range(min(_NBUF - 1, n_chunks)):
            hin[ci] = issue_in(ci)
        for ci in range(n_chunks):
            p = ci % _NBUF
            hx, hp = hin[ci]
            hx.wait()
            hp.wait()
            compute(p)
            r0 = base + ci * CS
            hout[ci] = pltpu.async_copy(xbufs[p], out_hbm.at[pl.ds(r0, CS)], so[p])
            nxt = ci + _NBUF - 1
            if nxt < n_chunks:
                if nxt - _NBUF >= 0:
                    hout[nxt - _NBUF].wait()
                hin[nxt] = issue_in(nxt)
        for ci in range(max(0, n_chunks - _NBUF), n_chunks):
            hout[ci].wait()

    return k(x, pe)


# speedup vs baseline: 1.0460x; 1.0259x over previous
import functools
import jax
import jax.numpy as jnp
from jax import lax
from jax.experimental import pallas as pl
from jax.experimental.pallas import tpu as pltpu
from jax.experimental.pallas import tpu_sc as plsc

_NC = 2
_NS = 16
_NW = _NC * _NS
_L = 16
_NBUF = 6


def kernel(x, pos_embed_weight):
    S, B, D = x.shape
    pe = pos_embed_weight[:S]
    rows_per_w = S // _NW
    CS = 4
    n_chunks = rows_per_w // CS
    nvec = D // _L

    mesh = plsc.VectorSubcoreMesh(core_axis_name="c", subcore_axis_name="s")

    scratch = (
        [pltpu.VMEM((CS, B, D), jnp.float32) for _ in range(_NBUF)]
        + [pltpu.VMEM((CS, D), jnp.float32) for _ in range(_NBUF)]
        + [pltpu.SemaphoreType.DMA for _ in range(3 * _NBUF)]
    )

    @functools.partial(
        pl.kernel,
        mesh=mesh,
        out_type=jax.ShapeDtypeStruct((S, B, D), jnp.float32),
        scratch_types=scratch,
    )
    def k(x_hbm, pe_hbm, out_hbm, *bufs):
        xbufs = bufs[0:_NBUF]
        pbufs = bufs[_NBUF:2 * _NBUF]
        six = bufs[2 * _NBUF:3 * _NBUF]
        sip = bufs[3 * _NBUF:4 * _NBUF]
        so = bufs[4 * _NBUF:5 * _NBUF]
        wid = lax.axis_index("s") * _NC + lax.axis_index("c")
        base = wid * rows_per_w

        def issue_in(ci):
            p = ci % _NBUF
            r0 = base + ci * CS
            hx = pltpu.async_copy(x_hbm.at[pl.ds(r0, CS)], xbufs[p], six[p])
            hp = pltpu.async_copy(pe_hbm.at[pl.ds(r0, CS)], pbufs[p], sip[p])
            return hx, hp

        def compute(p):
            xb_ = xbufs[p]
            pb_ = pbufs[p]

            def row_body(r, _):
                @plsc.parallel_loop(0, nvec, unroll=8)
                def vec_body(j):
                    sl = pl.ds(j * _L, _L)
                    pev = pb_[r, sl]
                    for b in range(B):
                        xb_[r, b, sl] = xb_[r, b, sl] + pev

                return 0

            lax.fori_loop(0, CS, row_body, 0)

        hin = {}
        hout = {}
        for ci in range(min(_NBUF - 1, n_chunks)):
            hin[ci] = issue_in(ci)
        for ci in range(n_chunks):
            p = ci % _NBUF
            hx, hp = hin[ci]
            hx.wait()
            hp.wait()
            compute(p)
            r0 = base + ci * CS
            hout[ci] = pltpu.async_copy(xbufs[p], out_hbm.at[pl.ds(r0, CS)], so[p])
            nxt = ci + _NBUF - 1
            if nxt < n_chunks:
                if nxt - _NBUF >= 0:
                    hout[nxt - _NBUF].wait()
                hin[nxt] = issue_in(nxt)
        for ci in range(max(0, n_chunks - _NBUF), n_chunks):
            hout[ci].wait()

    return k(x, pe)


# speedup vs baseline: 1.0494x; 1.0033x over previous
import functools
import jax
import jax.numpy as jnp
from jax import lax
from jax.experimental import pallas as pl
from jax.experimental.pallas import tpu as pltpu
from jax.experimental.pallas import tpu_sc as plsc

_NC = 2
_NS = 16
_NW = _NC * _NS
_L = 16
_NBUF = 3


def kernel(x, pos_embed_weight):
    S, B, D = x.shape
    pe = pos_embed_weight[:S]
    rows_per_w = S // _NW
    CS = 8
    n_chunks = rows_per_w // CS
    nvec = D // _L

    mesh = plsc.VectorSubcoreMesh(core_axis_name="c", subcore_axis_name="s")

    scratch = (
        [pltpu.VMEM((CS, B, D), jnp.float32) for _ in range(_NBUF)]
        + [pltpu.VMEM((CS, D), jnp.float32) for _ in range(_NBUF)]
        + [pltpu.SemaphoreType.DMA for _ in range(3 * _NBUF)]
    )

    @functools.partial(
        pl.kernel,
        mesh=mesh,
        out_type=jax.ShapeDtypeStruct((S, B, D), jnp.float32),
        scratch_types=scratch,
    )
    def k(x_hbm, pe_hbm, out_hbm, *bufs):
        xbufs = bufs[0:_NBUF]
        pbufs = bufs[_NBUF:2 * _NBUF]
        six = bufs[2 * _NBUF:3 * _NBUF]
        sip = bufs[3 * _NBUF:4 * _NBUF]
        so = bufs[4 * _NBUF:5 * _NBUF]
        wid = lax.axis_index("s") * _NC + lax.axis_index("c")
        base = wid * rows_per_w

        def issue_in(ci):
            p = ci % _NBUF
            r0 = base + ci * CS
            hx = pltpu.async_copy(x_hbm.at[pl.ds(r0, CS)], xbufs[p], six[p])
            hp = pltpu.async_copy(pe_hbm.at[pl.ds(r0, CS)], pbufs[p], sip[p])
            return hx, hp

        def compute(p):
            xb_ = xbufs[p]
            pb_ = pbufs[p]

            def row_body(r, _):
                @plsc.parallel_loop(0, nvec, unroll=8)
                def vec_body(j):
                    sl = pl.ds(j * _L, _L)
                    pev = pb_[r, sl]
                    for b in range(B):
                        xb_[r, b, sl] = xb_[r, b, sl] + pev

                return 0

            lax.fori_loop(0, CS, row_body, 0)

        hin = {}
        hout = {}
        for ci in range(min(_NBUF - 1, n_chunks)):
            hin[ci] = issue_in(ci)
        for ci in range(n_chunks):
            p = ci % _NBUF
            hx, hp = hin[ci]
            hx.wait()
            hp.wait()
            compute(p)
            r0 = base + ci * CS
            hout[ci] = pltpu.async_copy(xbufs[p], out_hbm.at[pl.ds(r0, CS)], so[p])
            nxt = ci + _NBUF - 1
            if nxt < n_chunks:
                if nxt - _NBUF >= 0:
                    hout[nxt - _NBUF].wait()
                hin[nxt] = issue_in(nxt)
        for ci in range(max(0, n_chunks - _NBUF), n_chunks):
            hout[ci].wait()

    return k(x, pe)
